# baseline (speedup 1.0000x reference)
import jax
import jax.numpy as jnp
from jax.experimental import pallas as pl
from jax.experimental.pallas import tpu as pltpu

_B, _N, _C = 128, 25, 128
_J = 25
_HID = 32
_K = 8
_E = _J * _J


def _edge_attn_body(spd_ref, src_ref, w1t_ref, a_ref, w2t_ref, out_ref, d_ref):
    @pl.when(pl.program_id(0) == 0)
    def _build_d():
        spd = spd_ref[...]
        n_iota = jax.lax.broadcasted_iota(jnp.int32, (_E, _N), 1)
        d = jnp.zeros((_E, _N), dtype=jnp.float32)
        for k in range(_K - 1):
            ends_k = spd[:, k][:, None]
            heads_k = spd[:, k][:, None]
            d = d + (ends_k == n_iota).astype(jnp.float32)
            d = d - (heads_k == n_iota).astype(jnp.float32)
        d_ref[...] = d

    d = d_ref[...]
    src_b = src_ref[0]
    p = jnp.dot(src_b, w1t_ref[...], preferred_element_type=jnp.float32)
    h = jnp.dot(d, p, preferred_element_type=jnp.float32)
    alpha = a_ref[0, 0]
    h = jnp.maximum(h, 0.0) + alpha * jnp.minimum(h, 0.0)
    o = jnp.dot(h, w2t_ref[...], preferred_element_type=jnp.float32)
    out_ref[0] = o


def kernel(src, s_SPD, W1, a, W2):
    spd = s_SPD.reshape(_E, _K)
    w1t = W1.T
    w2t = W2.T
    a2 = a.reshape(1, 1)
    out = pl.pallas_call(
        _edge_attn_body,
        grid=(_B,),
        in_specs=[
            pl.BlockSpec((_E, _K), lambda b: (0, 0)),
            pl.BlockSpec((1, _N, _C), lambda b: (b, 0, 0)),
            pl.BlockSpec((_C, _HID), lambda b: (0, 0)),
            pl.BlockSpec((1, 1), lambda b: (0, 0)),
            pl.BlockSpec((_HID, 1), lambda b: (0, 0)),
        ],
        out_specs=pl.BlockSpec((1, _E, 1), lambda b: (b, 0, 0)),
        out_shape=jax.ShapeDtypeStruct((_B, _E, 1), jnp.float32),
        scratch_shapes=[pltpu.VMEM((_E, _N), jnp.float32)],
    )(spd, src, w1t, a2, w2t)
    return out.reshape(_B, _J, _J, 1)


# speedup vs baseline: 1.9422x; 1.9422x over previous
import jax
import jax.numpy as jnp
from jax.experimental import pallas as pl

_B, _N, _C = 128, 25, 128
_J = 25
_HID = 32
_K = 8
_E = _J * _J


def _edge_attn_body(spdt_ref, srct_ref, w1_ref, a_ref, w2_ref, out_ref):
    spdt = spdt_ref[...]
    n_iota = jax.lax.broadcasted_iota(jnp.int32, (_N, _E), 0)
    dt = jnp.zeros((_N, _E), dtype=jnp.float32)
    for k in range(_K - 1):
        ends_k = spdt[k][None, :]
        heads_k = spdt[k][None, :]
        dt = dt + (ends_k == n_iota).astype(jnp.float32)
        dt = dt - (heads_k == n_iota).astype(jnp.float32)

    w1 = w1_ref[...]
    w2 = w2_ref[...]
    alpha = a_ref[0, 0]

    def per_batch(b, carry):
        pt = jnp.dot(w1, srct_ref[b], preferred_element_type=jnp.float32)
        ht = jnp.dot(pt, dt, preferred_element_type=jnp.float32)
        ht = jnp.maximum(ht, 0.0) + alpha * jnp.minimum(ht, 0.0)
        ot = jnp.dot(w2, ht, preferred_element_type=jnp.float32)
        out_ref[b] = ot
        return carry

    jax.lax.fori_loop(0, _B, per_batch, 0)


def kernel(src, s_SPD, W1, a, W2):
    spdt = s_SPD.reshape(_E, _K).T
    srct = src.transpose(0, 2, 1)
    a2 = a.reshape(1, 1)
    out = pl.pallas_call(
        _edge_attn_body,
        in_specs=[
            pl.BlockSpec((_K, _E), lambda: (0, 0)),
            pl.BlockSpec((_B, _C, _N), lambda: (0, 0, 0)),
            pl.BlockSpec((_HID, _C), lambda: (0, 0)),
            pl.BlockSpec((1, 1), lambda: (0, 0)),
            pl.BlockSpec((1, _HID), lambda: (0, 0)),
        ],
        out_specs=pl.BlockSpec((_B, 1, _E), lambda: (0, 0, 0)),
        out_shape=jax.ShapeDtypeStruct((_B, 1, _E), jnp.float32),
    )(spdt, srct, W1, a2, W2)
    return out.reshape(_B, _J, _J, 1)


# speedup vs baseline: 5.0911x; 2.6213x over previous
import jax
import jax.numpy as jnp
from jax.experimental import pallas as pl

_B, _N, _C = 128, 25, 128
_J = 25
_HID = 32
_K = 8
_E = _J * _J


_U = 8


def _edge_attn_body(spdt_ref, srct_ref, w1_ref, a_ref, w2blk_ref, out_ref):
    spdt = spdt_ref[...]
    n_iota = jax.lax.broadcasted_iota(jnp.int32, (_N, _E), 0)
    dt = jnp.zeros((_N, _E), dtype=jnp.float32)
    for k in range(_K - 1):
        ends_k = spdt[k][None, :]
        heads_k = spdt[k][None, :]
        dt = dt + (ends_k == n_iota).astype(jnp.float32)
        dt = dt - (heads_k == n_iota).astype(jnp.float32)

    w1 = w1_ref[...]
    w2blk = w2blk_ref[...]
    alpha = a_ref[0, 0]

    def per_group(i, carry):
        base = i * _U
        pts = [
            jnp.dot(w1, srct_ref[base + j], preferred_element_type=jnp.float32)
            for j in range(_U)
        ]
        pcat = jnp.concatenate(pts, axis=0)
        hcat = jnp.dot(pcat, dt, preferred_element_type=jnp.float32)
        hcat = jnp.maximum(hcat, 0.0) + alpha * jnp.minimum(hcat, 0.0)
        og = jnp.dot(w2blk, hcat, preferred_element_type=jnp.float32)
        out_ref[pl.ds(base, _U), :] = og
        return carry

    jax.lax.fori_loop(0, _B // _U, per_group, 0)


def kernel(src, s_SPD, W1, a, W2):
    spdt = s_SPD.reshape(_E, _K).T
    srct = src.transpose(0, 2, 1)
    a2 = a.reshape(1, 1)
    w2blk = jnp.kron(jnp.eye(_U, dtype=jnp.float32), W2)
    out = pl.pallas_call(
        _edge_attn_body,
        in_specs=[
            pl.BlockSpec((_K, _E), lambda: (0, 0)),
            pl.BlockSpec((_B, _C, _N), lambda: (0, 0, 0)),
            pl.BlockSpec((_HID, _C), lambda: (0, 0)),
            pl.BlockSpec((1, 1), lambda: (0, 0)),
            pl.BlockSpec((_U, _U * _HID), lambda: (0, 0)),
        ],
        out_specs=pl.BlockSpec((_B, _E), lambda: (0, 0)),
        out_shape=jax.ShapeDtypeStruct((_B, _E), jnp.float32),
    )(spdt, srct, W1, a2, w2blk)
    return out.reshape(_B, _J, _J, 1)


# speedup vs baseline: 5.8106x; 1.1413x over previous
import jax
import jax.numpy as jnp
from jax.experimental import pallas as pl

_B, _N, _C = 128, 25, 128
_J = 25
_HID = 32
_K = 8
_E = _J * _J


_U = 16


def _edge_attn_body(spdt_ref, srct_ref, w1_ref, a_ref, w2blk_ref, out_ref):
    spdt = spdt_ref[...]
    n_iota = jax.lax.broadcasted_iota(jnp.int32, (_N, _E), 0)
    dt = jnp.zeros((_N, _E), dtype=jnp.float32)
    for k in range(_K - 1):
        ends_k = spdt[k][None, :]
        heads_k = spdt[k][None, :]
        dt = dt + (ends_k == n_iota).astype(jnp.float32)
        dt = dt - (heads_k == n_iota).astype(jnp.float32)

    w1 = w1_ref[...]
    w2blk = w2blk_ref[...]
    alpha = a_ref[0, 0]

    def per_group(i, carry):
        base = i * _U
        pts = [
            jnp.dot(w1, srct_ref[base + j], preferred_element_type=jnp.float32)
            for j in range(_U)
        ]
        pcat = jnp.concatenate(pts, axis=0)
        hcat = jnp.dot(pcat, dt, preferred_element_type=jnp.float32)
        hcat = jnp.maximum(hcat, 0.0) + alpha * jnp.minimum(hcat, 0.0)
        og = jnp.dot(w2blk, hcat, preferred_element_type=jnp.float32)
        out_ref[pl.ds(base, _U), :] = og
        return carry

    jax.lax.fori_loop(0, _B // _U, per_group, 0)


def kernel(src, s_SPD, W1, a, W2):
    spdt = s_SPD.reshape(_E, _K).T
    srct = src.transpose(0, 2, 1)
    a2 = a.reshape(1, 1)
    w2blk = jnp.kron(jnp.eye(_U, dtype=jnp.float32), W2)
    out = pl.pallas_call(
        _edge_attn_body,
        in_specs=[
            pl.BlockSpec((_K, _E), lambda: (0, 0)),
            pl.BlockSpec((_B, _C, _N), lambda: (0, 0, 0)),
            pl.BlockSpec((_HID, _C), lambda: (0, 0)),
            pl.BlockSpec((1, 1), lambda: (0, 0)),
            pl.BlockSpec((_U, _U * _HID), lambda: (0, 0)),
        ],
        out_specs=pl.BlockSpec((_B, _E), lambda: (0, 0)),
        out_shape=jax.ShapeDtypeStruct((_B, _E), jnp.float32),
    )(spdt, srct, W1, a2, w2blk)
    return out.reshape(_B, _J, _J, 1)


# speedup vs baseline: 5.9294x; 1.0205x over previous
import jax
import jax.numpy as jnp
from jax.experimental import pallas as pl

_B, _N, _C = 128, 25, 128
_J = 25
_HID = 32
_K = 8
_E = _J * _J


_U = 32


def _edge_attn_body(spdt_ref, srct_ref, w1_ref, a_ref, w2blk_ref, out_ref):
    spdt = spdt_ref[...]
    n_iota = jax.lax.broadcasted_iota(jnp.int32, (_N, _E), 0)
    dt = jnp.zeros((_N, _E), dtype=jnp.float32)
    for k in range(_K - 1):
        ends_k = spdt[k][None, :]
        heads_k = spdt[k][None, :]
        dt = dt + (ends_k == n_iota).astype(jnp.float32)
        dt = dt - (heads_k == n_iota).astype(jnp.float32)

    w1 = w1_ref[...]
    w2blk = w2blk_ref[...]
    alpha = a_ref[0, 0]

    def per_group(i, carry):
        base = i * _U
        pts = [
            jnp.dot(w1, srct_ref[base + j], preferred_element_type=jnp.float32)
            for j in range(_U)
        ]
        pcat = jnp.concatenate(pts, axis=0)
        hcat = jnp.dot(pcat, dt, preferred_element_type=jnp.float32)
        hcat = jnp.maximum(hcat, 0.0) + alpha * jnp.minimum(hcat, 0.0)
        og = jnp.dot(w2blk, hcat, preferred_element_type=jnp.float32)
        out_ref[pl.ds(base, _U), :] = og
        return carry

    jax.lax.fori_loop(0, _B // _U, per_group, 0)


def kernel(src, s_SPD, W1, a, W2):
    spdt = s_SPD.reshape(_E, _K).T
    srct = src.transpose(0, 2, 1)
    a2 = a.reshape(1, 1)
    w2blk = jnp.kron(jnp.eye(_U, dtype=jnp.float32), W2)
    out = pl.pallas_call(
        _edge_attn_body,
        in_specs=[
            pl.BlockSpec((_K, _E), lambda: (0, 0)),
            pl.BlockSpec((_B, _C, _N), lambda: (0, 0, 0)),
            pl.BlockSpec((_HID, _C), lambda: (0, 0)),
            pl.BlockSpec((1, 1), lambda: (0, 0)),
            pl.BlockSpec((_U, _U * _HID), lambda: (0, 0)),
        ],
        out_specs=pl.BlockSpec((_B, _E), lambda: (0, 0)),
        out_shape=jax.ShapeDtypeStruct((_B, _E), jnp.float32),
    )(spdt, srct, W1, a2, w2blk)
    return out.reshape(_B, _J, _J, 1)
